# baseline (speedup 1.0000x reference)
import functools

import jax
import jax.numpy as jnp
from jax import lax
from jax.experimental import pallas as pl
from jax.experimental.pallas import tpu as pltpu
from jax.experimental.pallas import tpu_sc as plsc

F = 50000
L = 10000
T = 8
K = 64
D = 64

NC = 2
NS = 16
NW = NC * NS
CH = 128
NBUF_G = 7
NBUF = 4
SEG_CH = 2

F_pad = 50176
L_pad = 10240
SEG_W = L_pad // NW

MM_SCALE = (1e-09, 0.0001, 0.001, 1.0, 1.0)


def _worker_id():
    return lax.axis_index("s") * NC + lax.axis_index("c")


def sc_gather(table, idx, d):
    b = idx.shape[0]
    b_per_w = b // NW
    nch = b_per_w // CH
    mesh = plsc.VectorSubcoreMesh(core_axis_name="c", subcore_axis_name="s", num_cores=NC, num_subcores=NS)

    @functools.partial(
        pl.kernel, mesh=mesh,
        compiler_params=pltpu.CompilerParams(use_tc_tiling_on_sc=False),
        out_type=jax.ShapeDtypeStruct((b, d), jnp.float32),
        scratch_types=[
            pltpu.VMEM((b_per_w,), jnp.int32),
            pltpu.VMEM((NBUF_G, CH, d), jnp.float32),
        ] + [pltpu.SemaphoreType.DMA] * NBUF_G,
    )
    def k(table_hbm, idx_hbm, out_hbm, idx_v, rows_v, *sems):
        base = _worker_id() * b_per_w
        pltpu.sync_copy(idx_hbm.at[pl.ds(base, b_per_w)], idx_v)

        def gather(c, buf):
            pltpu.async_copy(
                table_hbm.at[idx_v.at[pl.ds(c * CH, CH)]],
                rows_v.at[buf], sems[buf])

        for b in range(NBUF_G):
            gather(b, b)

        def step(g, carry):
            for b in range(NBUF_G):
                c = g * NBUF_G + b
                pltpu.make_async_copy(
                    table_hbm.at[idx_v.at[pl.ds(c * CH, CH)]],
                    rows_v.at[b], sems[b]).wait()
                pltpu.async_copy(
                    rows_v.at[b], out_hbm.at[pl.ds(base + c * CH, CH)],
                    sems[b]).wait()

                @pl.when(c + NBUF_G < nch)
                def _():
                    gather(c + NBUF_G, b)
            return carry

        lax.fori_loop(0, nch // NBUF_G, step, 0)

    return k(table, idx)


def sc_gather_sum(table, idx, d):
    mesh = plsc.VectorSubcoreMesh(core_axis_name="c", subcore_axis_name="s", num_cores=NC, num_subcores=NS)
    idx_per_w = SEG_W * K
    nvec = d // 16

    @functools.partial(
        pl.kernel, mesh=mesh,
        compiler_params=pltpu.CompilerParams(use_tc_tiling_on_sc=False),
        out_type=jax.ShapeDtypeStruct((L_pad, d), jnp.float32),
        scratch_types=[
            pltpu.VMEM((idx_per_w,), jnp.int32),
            pltpu.VMEM((NBUF, SEG_CH * K, d), jnp.float32),
            pltpu.VMEM((SEG_W, d), jnp.float32),
            pltpu.SemaphoreType.DMA,
            pltpu.SemaphoreType.DMA,
            pltpu.SemaphoreType.DMA,
            pltpu.SemaphoreType.DMA,
        ],
    )
    def k(table_hbm, idx_hbm, out_hbm, idx_v, rows_v, acc_v, s0, s1, s2, s3):
        sems = (s0, s1, s2, s3)
        wid = _worker_id()
        pltpu.sync_copy(idx_hbm.at[pl.ds(wid * idx_per_w, idx_per_w)], idx_v)
        nch_s = SEG_W // SEG_CH

        def gather(c, buf):
            pltpu.async_copy(
                table_hbm.at[idx_v.at[pl.ds(c * SEG_CH * K, SEG_CH * K)]],
                rows_v.at[buf], sems[buf])

        for b in range(NBUF):
            gather(b, b)

        def step(g, carry):
            for b in range(NBUF):
                c = g * NBUF + b
                pltpu.make_async_copy(
                    table_hbm.at[idx_v.at[pl.ds(c * SEG_CH * K, SEG_CH * K)]],
                    rows_v.at[b], sems[b]).wait()
                for s in range(SEG_CH):
                    for j in range(nvec):
                        parts = [rows_v[b, s * K + p, pl.ds(j * 16, 16)]
                                 for p in range(4)]
                        for kk in range(4, K):
                            parts[kk % 4] = parts[kk % 4] + rows_v[
                                b, s * K + kk, pl.ds(j * 16, 16)]
                        acc_v[c * SEG_CH + s, pl.ds(j * 16, 16)] = (
                            (parts[0] + parts[1]) + (parts[2] + parts[3]))

                @pl.when(c + NBUF < nch_s)
                def _():
                    gather(c + NBUF, b)
            return carry

        lax.fori_loop(0, nch_s // NBUF, step, 0)
        pltpu.sync_copy(acc_v, out_hbm.at[pl.ds(wid * SEG_W, SEG_W)])

    return k(table, idx)


def _gelu(x):
    return 0.5 * x * (1.0 + lax.erf(x * 0.7071067811865476))


def tc_path_embed(pf, W1, b1, W2, b2):
    def body(pf_ref, w1_ref, b1_ref, w2_ref, b2_ref, out_ref):
        h = _gelu(jnp.dot(pf_ref[...], w1_ref[...],
                          preferred_element_type=jnp.float32) + b1_ref[...])
        out_ref[...] = _gelu(jnp.dot(h, w2_ref[...],
                                     preferred_element_type=jnp.float32) + b2_ref[...])

    bf = 6272
    grid = (F_pad // bf,)
    return pl.pallas_call(
        body,
        grid=grid,
        in_specs=[
            pl.BlockSpec((bf, 5), lambda i: (i, 0)),
            pl.BlockSpec((5, D), lambda i: (0, 0)),
            pl.BlockSpec((1, D), lambda i: (0, 0)),
            pl.BlockSpec((D, D), lambda i: (0, 0)),
            pl.BlockSpec((1, D), lambda i: (0, 0)),
        ],
        out_specs=pl.BlockSpec((bf, D), lambda i: (i, 0)),
        out_shape=jax.ShapeDtypeStruct((F_pad, D), jnp.float32),
    )(pf, W1, b1, W2, b2)


def tc_link_embed(cap, load_col, W1, b1, W2, b2):
    def body(cap_ref, load_ref, w1_ref, b1_ref, w2_ref, b2_ref, out_ref):
        c = cap_ref[...]
        f0 = c * 1e-05
        f1 = load_ref[...] / (c * 1e9)
        h = _gelu(f0 * w1_ref[0, :] + f1 * w1_ref[1, :] + b1_ref[...])
        out_ref[...] = _gelu(jnp.dot(h, w2_ref[...],
                                     preferred_element_type=jnp.float32) + b2_ref[...])

    return pl.pallas_call(
        body,
        grid=(1,),
        in_specs=[
            pl.BlockSpec((L_pad, 1), lambda i: (0, 0)),
            pl.BlockSpec((L_pad, 1), lambda i: (0, 0)),
            pl.BlockSpec((2, D), lambda i: (0, 0)),
            pl.BlockSpec((1, D), lambda i: (0, 0)),
            pl.BlockSpec((D, D), lambda i: (0, 0)),
            pl.BlockSpec((1, D), lambda i: (0, 0)),
        ],
        out_specs=pl.BlockSpec((L_pad, D), lambda i: (0, 0)),
        out_shape=jax.ShapeDtypeStruct((L_pad, D), jnp.float32),
    )(cap, load_col, W1, b1, W2, b2)


def _gru_math(h, xz, xr, xh, hz, hr, hh, bz, br_, bih, brh):
    z = jax.nn.sigmoid(xz + hz + bz)
    r = jax.nn.sigmoid(xr + hr + br_)
    cand = jnp.tanh(xh + bih + r * (hh + brh))
    return z * h + (1.0 - z) * cand


def tc_gru_scan(xs3, h0, Wzr, Whh, bs):
    bz, br_, bih, brh = bs

    def body(xs_ref, h0_ref, wzr, whh, bz_r, br_r, bih_r, brh_r,
             out_ref, hl_ref):
        h = h0_ref[...]
        out_ref[0] = h
        for t in range(T):
            cat = jnp.concatenate([xs_ref[t], h], axis=1)
            zr = jnp.dot(cat, wzr[...], preferred_element_type=jnp.float32)
            xhh = jnp.dot(cat, whh[...], preferred_element_type=jnp.float32)
            z = jax.nn.sigmoid(zr[:, :D] + bz_r[...])
            r = jax.nn.sigmoid(zr[:, D:] + br_r[...])
            cand = jnp.tanh(xhh[:, :D] + bih_r[...] + r * (xhh[:, D:] + brh_r[...]))
            h = z * h + (1.0 - z) * cand
            out_ref[t + 1] = h
        hl_ref[...] = h

    bf = 1568
    grid = (F_pad // bf,)
    wspec = pl.BlockSpec((2 * D, 2 * D), lambda i: (0, 0))
    bspec = pl.BlockSpec((1, D), lambda i: (0, 0))
    return pl.pallas_call(
        body,
        grid=grid,
        in_specs=[
            pl.BlockSpec((T, bf, D), lambda i: (0, i, 0)),
            pl.BlockSpec((bf, D), lambda i: (i, 0)),
            wspec, wspec,
            bspec, bspec, bspec, bspec,
        ],
        out_specs=[
            pl.BlockSpec((T + 1, bf, D), lambda i: (0, i, 0)),
            pl.BlockSpec((bf, D), lambda i: (i, 0)),
        ],
        out_shape=[
            jax.ShapeDtypeStruct((T + 1, F_pad, D), jnp.float32),
            jax.ShapeDtypeStruct((F_pad, D), jnp.float32),
        ],
    )(xs3, h0, Wzr, Whh, *bs)


def tc_link_gru(ls, psum, Wzr, Whh, bs):
    def body(x_ref, h_ref, wzr, whh, bz_r, br_r, bih_r, brh_r, out_ref):
        h = h_ref[...]
        cat = jnp.concatenate([x_ref[...], h], axis=1)
        zr = jnp.dot(cat, wzr[...], preferred_element_type=jnp.float32)
        xhh = jnp.dot(cat, whh[...], preferred_element_type=jnp.float32)
        z = jax.nn.sigmoid(zr[:, :D] + bz_r[...])
        r = jax.nn.sigmoid(zr[:, D:] + br_r[...])
        cand = jnp.tanh(xhh[:, :D] + bih_r[...] + r * (xhh[:, D:] + brh_r[...]))
        out_ref[...] = z * h + (1.0 - z) * cand

    bl = L_pad // 4
    wspec = pl.BlockSpec((2 * D, 2 * D), lambda i: (0, 0))
    bspec = pl.BlockSpec((1, D), lambda i: (0, 0))
    return pl.pallas_call(
        body,
        grid=(L_pad // bl,),
        in_specs=[
            pl.BlockSpec((bl, D), lambda i: (i, 0)),
            pl.BlockSpec((bl, D), lambda i: (i, 0)),
            wspec, wspec,
            bspec, bspec, bspec, bspec,
        ],
        out_specs=pl.BlockSpec((bl, D), lambda i: (i, 0)),
        out_shape=jax.ShapeDtypeStruct((L_pad, D), jnp.float32),
    )(psum, ls, Wzr, Whh, *bs)


def _softplus(x):
    return jnp.maximum(x, 0.0) + jnp.log(1.0 + jnp.exp(-jnp.abs(x)))


def tc_readout(pss, capg, W1, b1, W2, b2, W3, b3):
    def body(pss_ref, cap_ref, w1, b1r, w2, b2r, w3, b3r, out_ref):
        t = pl.program_id(1)
        x = pss_ref[0]
        h1 = _gelu(jnp.dot(x, w1[...], preferred_element_type=jnp.float32) + b1r[...])
        h2 = _gelu(jnp.dot(h1, w2[...], preferred_element_type=jnp.float32) + b2r[...])
        occ = _softplus(jnp.dot(h2, w3[...], preferred_element_type=jnp.float32) + b3r[...])
        contrib = occ / cap_ref[0][:, 0:1]

        @pl.when(t == 0)
        def _():
            out_ref[...] = contrib

        @pl.when(t != 0)
        def _():
            out_ref[...] += contrib

    bf = 3136
    return pl.pallas_call(
        body,
        grid=(F_pad // bf, T),
        in_specs=[
            pl.BlockSpec((1, bf, D), lambda i, t: (t + 1, i, 0)),
            pl.BlockSpec((1, bf, 16), lambda i, t: (t, i, 0)),
            pl.BlockSpec((D, D // 2), lambda i, t: (0, 0)),
            pl.BlockSpec((1, D // 2), lambda i, t: (0, 0)),
            pl.BlockSpec((D // 2, D // 4), lambda i, t: (0, 0)),
            pl.BlockSpec((1, D // 4), lambda i, t: (0, 0)),
            pl.BlockSpec((D // 4, 1), lambda i, t: (0, 0)),
            pl.BlockSpec((1, 1), lambda i, t: (0, 0)),
        ],
        out_specs=pl.BlockSpec((bf, 1), lambda i, t: (i, 0)),
        out_shape=jax.ShapeDtypeStruct((F_pad, 1), jnp.float32),
    )(pss, capg, W1, b1, W2, b2, W3, b3)


def _split_gru(Kw, RK, bi, br):
    z = jnp.zeros((D, D), jnp.float32)
    Wzr = jnp.concatenate([Kw[:, :2 * D], RK[:, :2 * D]], axis=0)
    Whh = jnp.concatenate([
        jnp.concatenate([Kw[:, 2 * D:], z], axis=1),
        jnp.concatenate([z, RK[:, 2 * D:]], axis=1)], axis=0)
    bs = ((bi[0:D] + br[0:D]).reshape(1, D),
          (bi[D:2 * D] + br[D:2 * D]).reshape(1, D),
          bi[2 * D:].reshape(1, D),
          br[2 * D:].reshape(1, D))
    return Wzr, Whh, bs


def kernel(flow_traffic, flow_packets, flow_packet_size, link_capacity,
           ipg_mean, ipg_var, link_to_path, path_to_link,
           pe_W1, pe_b1, pe_W2, pe_b2, le_W1, le_b1, le_W2, le_b2,
           pg_K, pg_RK, pg_bi, pg_br, lg_K, lg_RK, lg_bi, lg_br,
           ro_W1, ro_b1, ro_W2, ro_b2, ro_W3, ro_b3):
    lt = link_to_path.astype(jnp.int32)
    idx_link = jnp.pad(lt.T, ((0, 0), (0, F_pad - F))).reshape(-1)

    p2l = path_to_link.astype(jnp.int32)
    flow = jnp.pad(p2l[:, :, 0], ((0, L_pad - L), (0, 0)))
    pos = jnp.pad(p2l[:, :, 1], ((0, L_pad - L), (0, 0)))
    idx_ps = (pos * F_pad + flow).reshape(-1)
    idx_traffic = flow.reshape(-1)

    traffic_tbl = jnp.broadcast_to(flow_traffic, (F, 16))
    cap_tbl = jnp.broadcast_to(link_capacity, (L, 16))

    load_sum = sc_gather_sum(traffic_tbl, idx_traffic, 16)

    scale = jnp.array(MM_SCALE, jnp.float32)[:, None]
    pf = jnp.pad(jnp.concatenate([flow_traffic, flow_packets, flow_packet_size,
                                  ipg_mean, ipg_var], axis=1),
                 ((0, F_pad - F), (0, 0)))
    path_state = tc_path_embed(pf, pe_W1 * scale, pe_b1.reshape(1, D),
                               pe_W2, pe_b2.reshape(1, D))

    cap_pad = jnp.pad(link_capacity, ((0, L_pad - L), (0, 0)),
                      constant_values=1.0)
    link_state = tc_link_embed(cap_pad, load_sum[:, 0:1],
                               le_W1, le_b1.reshape(1, D),
                               le_W2, le_b2.reshape(1, D))

    capg = sc_gather(cap_tbl, idx_link, 16).reshape(T, F_pad, 16)

    pg_Wzr, pg_Whh, pg_bs = _split_gru(pg_K, pg_RK, pg_bi, pg_br)
    lg_Wzr, lg_Whh, lg_bs = _split_gru(lg_K, lg_RK, lg_bi, lg_br)

    pss = None
    for _ in range(4):
        xs = sc_gather(link_state, idx_link, D)
        pss, path_state = tc_gru_scan(xs.reshape(T, F_pad, D), path_state,
                                      pg_Wzr, pg_Whh, pg_bs)
        psum = sc_gather_sum(pss.reshape((T + 1) * F_pad, D), idx_ps, D)
        link_state = tc_link_gru(link_state, psum, lg_Wzr, lg_Whh, lg_bs)

    delay = tc_readout(pss, capg,
                       ro_W1, ro_b1.reshape(1, D // 2),
                       ro_W2, ro_b2.reshape(1, D // 4),
                       ro_W3, ro_b3.reshape(1, 1))
    return delay[:F]


# speedup vs baseline: 1.0085x; 1.0085x over previous
import functools

import jax
import jax.numpy as jnp
from jax import lax
from jax.experimental import pallas as pl
from jax.experimental.pallas import tpu as pltpu
from jax.experimental.pallas import tpu_sc as plsc

F = 50000
L = 10000
T = 8
K = 64
D = 64

NC = 2
NS = 16
NW = NC * NS
CH = 128
NBUF_G = 7
NBUF = 4
SEG_CH = 2

F_pad = 50176
L_pad = 10240
SEG_W = L_pad // NW

MM_SCALE = (1e-09, 0.0001, 0.001, 1.0, 1.0)


def _worker_id():
    return lax.axis_index("s") * NC + lax.axis_index("c")


def sc_gather(table, idx, d):
    b = idx.shape[0]
    b_per_w = b // NW
    nch = b_per_w // CH
    mesh = plsc.VectorSubcoreMesh(core_axis_name="c", subcore_axis_name="s", num_cores=NC, num_subcores=NS)

    @functools.partial(
        pl.kernel, mesh=mesh,
        compiler_params=pltpu.CompilerParams(use_tc_tiling_on_sc=False),
        out_type=jax.ShapeDtypeStruct((b, d), jnp.float32),
        scratch_types=[
            pltpu.VMEM((b_per_w,), jnp.int32),
            pltpu.VMEM((NBUF_G, CH, d), jnp.float32),
        ] + [pltpu.SemaphoreType.DMA] * NBUF_G,
    )
    def k(table_hbm, idx_hbm, out_hbm, idx_v, rows_v, *sems):
        base = _worker_id() * b_per_w
        pltpu.sync_copy(idx_hbm.at[pl.ds(base, b_per_w)], idx_v)

        def gather(c, buf):
            pltpu.async_copy(
                table_hbm.at[idx_v.at[pl.ds(c * CH, CH)]],
                rows_v.at[buf], sems[buf])

        for b in range(NBUF_G):
            gather(b, b)

        def step(g, carry):
            for b in range(NBUF_G):
                c = g * NBUF_G + b
                pltpu.make_async_copy(
                    table_hbm.at[idx_v.at[pl.ds(c * CH, CH)]],
                    rows_v.at[b], sems[b]).wait()
                pltpu.async_copy(
                    rows_v.at[b], out_hbm.at[pl.ds(base + c * CH, CH)],
                    sems[b]).wait()

                @pl.when(c + NBUF_G < nch)
                def _():
                    gather(c + NBUF_G, b)
            return carry

        lax.fori_loop(0, nch // NBUF_G, step, 0)

    return k(table, idx)


def sc_gather_sum(table, idx, d):
    mesh = plsc.VectorSubcoreMesh(core_axis_name="c", subcore_axis_name="s", num_cores=NC, num_subcores=NS)
    idx_per_w = SEG_W * K
    nvec = d // 16

    @functools.partial(
        pl.kernel, mesh=mesh,
        compiler_params=pltpu.CompilerParams(use_tc_tiling_on_sc=False),
        out_type=jax.ShapeDtypeStruct((L_pad, d), jnp.float32),
        scratch_types=[
            pltpu.VMEM((idx_per_w,), jnp.int32),
            pltpu.VMEM((NBUF, SEG_CH * K, d), jnp.float32),
            pltpu.VMEM((SEG_W, d), jnp.float32),
            pltpu.SemaphoreType.DMA,
            pltpu.SemaphoreType.DMA,
            pltpu.SemaphoreType.DMA,
            pltpu.SemaphoreType.DMA,
        ],
    )
    def k(table_hbm, idx_hbm, out_hbm, idx_v, rows_v, acc_v, s0, s1, s2, s3):
        sems = (s0, s1, s2, s3)
        wid = _worker_id()
        pltpu.sync_copy(idx_hbm.at[pl.ds(wid * idx_per_w, idx_per_w)], idx_v)
        nch_s = SEG_W // SEG_CH

        def gather(c, buf):
            pltpu.async_copy(
                table_hbm.at[idx_v.at[pl.ds(c * SEG_CH * K, SEG_CH * K)]],
                rows_v.at[buf], sems[buf])

        for b in range(NBUF):
            gather(b, b)

        def step(g, carry):
            for b in range(NBUF):
                c = g * NBUF + b
                pltpu.make_async_copy(
                    table_hbm.at[idx_v.at[pl.ds(c * SEG_CH * K, SEG_CH * K)]],
                    rows_v.at[b], sems[b]).wait()
                for s in range(SEG_CH):
                    for j in range(nvec):
                        parts = [rows_v[b, s * K + p, pl.ds(j * 16, 16)]
                                 for p in range(4)]
                        for kk in range(4, K):
                            parts[kk % 4] = parts[kk % 4] + rows_v[
                                b, s * K + kk, pl.ds(j * 16, 16)]
                        acc_v[c * SEG_CH + s, pl.ds(j * 16, 16)] = (
                            (parts[0] + parts[1]) + (parts[2] + parts[3]))

                @pl.when(c + NBUF < nch_s)
                def _():
                    gather(c + NBUF, b)
            return carry

        lax.fori_loop(0, nch_s // NBUF, step, 0)
        pltpu.sync_copy(acc_v, out_hbm.at[pl.ds(wid * SEG_W, SEG_W)])

    return k(table, idx)


def _gelu(x):
    return 0.5 * x * (1.0 + lax.erf(x * 0.7071067811865476))


def tc_path_embed(pf, W1, b1, W2, b2):
    def body(pf_ref, w1_ref, b1_ref, w2_ref, b2_ref, out_ref):
        h = _gelu(jnp.dot(pf_ref[...], w1_ref[...],
                          preferred_element_type=jnp.float32) + b1_ref[...])
        out_ref[...] = _gelu(jnp.dot(h, w2_ref[...],
                                     preferred_element_type=jnp.float32) + b2_ref[...])

    bf = 6272
    grid = (F_pad // bf,)
    return pl.pallas_call(
        body,
        grid=grid,
        in_specs=[
            pl.BlockSpec((bf, 5), lambda i: (i, 0)),
            pl.BlockSpec((5, D), lambda i: (0, 0)),
            pl.BlockSpec((1, D), lambda i: (0, 0)),
            pl.BlockSpec((D, D), lambda i: (0, 0)),
            pl.BlockSpec((1, D), lambda i: (0, 0)),
        ],
        out_specs=pl.BlockSpec((bf, D), lambda i: (i, 0)),
        out_shape=jax.ShapeDtypeStruct((F_pad, D), jnp.float32),
    )(pf, W1, b1, W2, b2)


def tc_link_embed(cap, load_col, W1, b1, W2, b2):
    def body(cap_ref, load_ref, w1_ref, b1_ref, w2_ref, b2_ref, out_ref):
        c = cap_ref[...]
        f0 = c * 1e-05
        f1 = load_ref[...] / (c * 1e9)
        h = _gelu(f0 * w1_ref[0, :] + f1 * w1_ref[1, :] + b1_ref[...])
        out_ref[...] = _gelu(jnp.dot(h, w2_ref[...],
                                     preferred_element_type=jnp.float32) + b2_ref[...])

    return pl.pallas_call(
        body,
        grid=(1,),
        in_specs=[
            pl.BlockSpec((L_pad, 1), lambda i: (0, 0)),
            pl.BlockSpec((L_pad, 1), lambda i: (0, 0)),
            pl.BlockSpec((2, D), lambda i: (0, 0)),
            pl.BlockSpec((1, D), lambda i: (0, 0)),
            pl.BlockSpec((D, D), lambda i: (0, 0)),
            pl.BlockSpec((1, D), lambda i: (0, 0)),
        ],
        out_specs=pl.BlockSpec((L_pad, D), lambda i: (0, 0)),
        out_shape=jax.ShapeDtypeStruct((L_pad, D), jnp.float32),
    )(cap, load_col, W1, b1, W2, b2)


def _sigmoid(x):
    return 0.5 * (1.0 + jnp.tanh(0.5 * x))


def tc_gru_scan(xs3, h0, Wzr, Whh, bs):
    bz, br_, bih, brh = bs

    def body(xs_ref, h0_ref, wzr, whh, bz_r, br_r, bih_r, brh_r,
             out_ref, hl_ref):
        h = h0_ref[...]
        out_ref[0] = h
        for t in range(T):
            cat = jnp.concatenate([xs_ref[t], h], axis=1)
            zr = jnp.dot(cat, wzr[...], preferred_element_type=jnp.float32)
            xhh = jnp.dot(cat, whh[...], preferred_element_type=jnp.float32)
            z = _sigmoid(zr[:, :D] + bz_r[...])
            r = _sigmoid(zr[:, D:] + br_r[...])
            cand = jnp.tanh(xhh[:, :D] + bih_r[...] + r * (xhh[:, D:] + brh_r[...]))
            h = z * h + (1.0 - z) * cand
            out_ref[t + 1] = h
        hl_ref[...] = h

    bf = 1568
    grid = (F_pad // bf,)
    wspec = pl.BlockSpec((2 * D, 2 * D), lambda i: (0, 0))
    bspec = pl.BlockSpec((1, D), lambda i: (0, 0))
    return pl.pallas_call(
        body,
        grid=grid,
        in_specs=[
            pl.BlockSpec((T, bf, D), lambda i: (0, i, 0)),
            pl.BlockSpec((bf, D), lambda i: (i, 0)),
            wspec, wspec,
            bspec, bspec, bspec, bspec,
        ],
        out_specs=[
            pl.BlockSpec((T + 1, bf, D), lambda i: (0, i, 0)),
            pl.BlockSpec((bf, D), lambda i: (i, 0)),
        ],
        out_shape=[
            jax.ShapeDtypeStruct((T + 1, F_pad, D), jnp.float32),
            jax.ShapeDtypeStruct((F_pad, D), jnp.float32),
        ],
    )(xs3, h0, Wzr, Whh, *bs)


def tc_link_gru(ls, psum, Wzr, Whh, bs):
    def body(x_ref, h_ref, wzr, whh, bz_r, br_r, bih_r, brh_r, out_ref):
        h = h_ref[...]
        cat = jnp.concatenate([x_ref[...], h], axis=1)
        zr = jnp.dot(cat, wzr[...], preferred_element_type=jnp.float32)
        xhh = jnp.dot(cat, whh[...], preferred_element_type=jnp.float32)
        z = _sigmoid(zr[:, :D] + bz_r[...])
        r = _sigmoid(zr[:, D:] + br_r[...])
        cand = jnp.tanh(xhh[:, :D] + bih_r[...] + r * (xhh[:, D:] + brh_r[...]))
        out_ref[...] = z * h + (1.0 - z) * cand

    bl = L_pad // 4
    wspec = pl.BlockSpec((2 * D, 2 * D), lambda i: (0, 0))
    bspec = pl.BlockSpec((1, D), lambda i: (0, 0))
    return pl.pallas_call(
        body,
        grid=(L_pad // bl,),
        in_specs=[
            pl.BlockSpec((bl, D), lambda i: (i, 0)),
            pl.BlockSpec((bl, D), lambda i: (i, 0)),
            wspec, wspec,
            bspec, bspec, bspec, bspec,
        ],
        out_specs=pl.BlockSpec((bl, D), lambda i: (i, 0)),
        out_shape=jax.ShapeDtypeStruct((L_pad, D), jnp.float32),
    )(psum, ls, Wzr, Whh, *bs)


def _softplus(x):
    return jnp.maximum(x, 0.0) + jnp.log(1.0 + jnp.exp(-jnp.abs(x)))


def tc_readout(pss, capg, W1, b1, W2, b2, W3, b3):
    def body(pss_ref, cap_ref, w1, b1r, w2, b2r, w3, b3r, out_ref):
        t = pl.program_id(1)
        x = pss_ref[0]
        h1 = _gelu(jnp.dot(x, w1[...], preferred_element_type=jnp.float32) + b1r[...])
        h2 = _gelu(jnp.dot(h1, w2[...], preferred_element_type=jnp.float32) + b2r[...])
        occ = _softplus(jnp.dot(h2, w3[...], preferred_element_type=jnp.float32) + b3r[...])
        contrib = occ / cap_ref[0][:, 0:1]

        @pl.when(t == 0)
        def _():
            out_ref[...] = contrib

        @pl.when(t != 0)
        def _():
            out_ref[...] += contrib

    bf = 3136
    return pl.pallas_call(
        body,
        grid=(F_pad // bf, T),
        in_specs=[
            pl.BlockSpec((1, bf, D), lambda i, t: (t + 1, i, 0)),
            pl.BlockSpec((1, bf, 16), lambda i, t: (t, i, 0)),
            pl.BlockSpec((D, D // 2), lambda i, t: (0, 0)),
            pl.BlockSpec((1, D // 2), lambda i, t: (0, 0)),
            pl.BlockSpec((D // 2, D // 4), lambda i, t: (0, 0)),
            pl.BlockSpec((1, D // 4), lambda i, t: (0, 0)),
            pl.BlockSpec((D // 4, 1), lambda i, t: (0, 0)),
            pl.BlockSpec((1, 1), lambda i, t: (0, 0)),
        ],
        out_specs=pl.BlockSpec((bf, 1), lambda i, t: (i, 0)),
        out_shape=jax.ShapeDtypeStruct((F_pad, 1), jnp.float32),
    )(pss, capg, W1, b1, W2, b2, W3, b3)


def _split_gru(Kw, RK, bi, br):
    z = jnp.zeros((D, D), jnp.float32)
    Wzr = jnp.concatenate([Kw[:, :2 * D], RK[:, :2 * D]], axis=0)
    Whh = jnp.concatenate([
        jnp.concatenate([Kw[:, 2 * D:], z], axis=1),
        jnp.concatenate([z, RK[:, 2 * D:]], axis=1)], axis=0)
    bs = ((bi[0:D] + br[0:D]).reshape(1, D),
          (bi[D:2 * D] + br[D:2 * D]).reshape(1, D),
          bi[2 * D:].reshape(1, D),
          br[2 * D:].reshape(1, D))
    return Wzr, Whh, bs


def kernel(flow_traffic, flow_packets, flow_packet_size, link_capacity,
           ipg_mean, ipg_var, link_to_path, path_to_link,
           pe_W1, pe_b1, pe_W2, pe_b2, le_W1, le_b1, le_W2, le_b2,
           pg_K, pg_RK, pg_bi, pg_br, lg_K, lg_RK, lg_bi, lg_br,
           ro_W1, ro_b1, ro_W2, ro_b2, ro_W3, ro_b3):
    lt = link_to_path.astype(jnp.int32)
    idx_link = jnp.pad(lt.T, ((0, 0), (0, F_pad - F))).reshape(-1)

    p2l = path_to_link.astype(jnp.int32)
    flow = jnp.pad(p2l[:, :, 0], ((0, L_pad - L), (0, 0)))
    pos = jnp.pad(p2l[:, :, 1], ((0, L_pad - L), (0, 0)))
    idx_ps = (pos * F_pad + flow).reshape(-1)
    idx_traffic = flow.reshape(-1)

    traffic_tbl = jnp.broadcast_to(flow_traffic, (F, 16))
    cap_tbl = jnp.broadcast_to(link_capacity, (L, 16))

    load_sum = sc_gather_sum(traffic_tbl, idx_traffic, 16)

    scale = jnp.array(MM_SCALE, jnp.float32)[:, None]
    pf = jnp.pad(jnp.concatenate([flow_traffic, flow_packets, flow_packet_size,
                                  ipg_mean, ipg_var], axis=1),
                 ((0, F_pad - F), (0, 0)))
    path_state = tc_path_embed(pf, pe_W1 * scale, pe_b1.reshape(1, D),
                               pe_W2, pe_b2.reshape(1, D))

    cap_pad = jnp.pad(link_capacity, ((0, L_pad - L), (0, 0)),
                      constant_values=1.0)
    link_state = tc_link_embed(cap_pad, load_sum[:, 0:1],
                               le_W1, le_b1.reshape(1, D),
                               le_W2, le_b2.reshape(1, D))

    capg = sc_gather(cap_tbl, idx_link, 16).reshape(T, F_pad, 16)

    pg_Wzr, pg_Whh, pg_bs = _split_gru(pg_K, pg_RK, pg_bi, pg_br)
    lg_Wzr, lg_Whh, lg_bs = _split_gru(lg_K, lg_RK, lg_bi, lg_br)

    pss = None
    for r in range(4):
        xs = sc_gather(link_state, idx_link, D)
        pss, path_state = tc_gru_scan(xs.reshape(T, F_pad, D), path_state,
                                      pg_Wzr, pg_Whh, pg_bs)
        if r < 3:
            psum = sc_gather_sum(pss.reshape((T + 1) * F_pad, D), idx_ps, D)
            link_state = tc_link_gru(link_state, psum, lg_Wzr, lg_Whh, lg_bs)

    delay = tc_readout(pss, capg,
                       ro_W1, ro_b1.reshape(1, D // 2),
                       ro_W2, ro_b2.reshape(1, D // 4),
                       ro_W3, ro_b3.reshape(1, 1))
    return delay[:F]


# speedup vs baseline: 1.3943x; 1.3826x over previous
import functools

import jax
import jax.numpy as jnp
from jax import lax
from jax.experimental import pallas as pl
from jax.experimental.pallas import tpu as pltpu
from jax.experimental.pallas import tpu_sc as plsc

F = 50000
L = 10000
T = 8
K = 64
D = 64

NC = 2
NS = 16
NW = NC * NS
CH = 128
NBUF_G = 7
NBUF = 4
SEG_CH = 2

F_pad = 50176
L_pad = 10240
SEG_W = L_pad // NW

MM_SCALE = (1e-09, 0.0001, 0.001, 1.0, 1.0)


def _worker_id():
    return lax.axis_index("s") * NC + lax.axis_index("c")


def sc_gather(table, idx, d):
    b = idx.shape[0]
    b_per_w = b // NW
    nch = b_per_w // CH
    mesh = plsc.VectorSubcoreMesh(core_axis_name="c", subcore_axis_name="s", num_cores=NC, num_subcores=NS)

    @functools.partial(
        pl.kernel, mesh=mesh,
        compiler_params=pltpu.CompilerParams(use_tc_tiling_on_sc=False),
        out_type=jax.ShapeDtypeStruct((b, d), jnp.float32),
        scratch_types=[
            pltpu.VMEM((b_per_w,), jnp.int32),
            pltpu.VMEM((NBUF_G, CH, d), jnp.float32),
        ] + [pltpu.SemaphoreType.DMA] * NBUF_G,
    )
    def k(table_hbm, idx_hbm, out_hbm, idx_v, rows_v, *sems):
        base = _worker_id() * b_per_w
        pltpu.sync_copy(idx_hbm.at[pl.ds(base, b_per_w)], idx_v)

        def gather(c, buf):
            pltpu.async_copy(
                table_hbm.at[idx_v.at[pl.ds(c * CH, CH)]],
                rows_v.at[buf], sems[buf])

        for b in range(NBUF_G):
            gather(b, b)

        def step(g, carry):
            for b in range(NBUF_G):
                c = g * NBUF_G + b
                pltpu.make_async_copy(
                    table_hbm.at[idx_v.at[pl.ds(c * CH, CH)]],
                    rows_v.at[b], sems[b]).wait()
                pltpu.async_copy(
                    rows_v.at[b], out_hbm.at[pl.ds(base + c * CH, CH)],
                    sems[b]).wait()

                @pl.when(c + NBUF_G < nch)
                def _():
                    gather(c + NBUF_G, b)
            return carry

        lax.fori_loop(0, nch // NBUF_G, step, 0)

    return k(table, idx)


def sc_gather_sum(table, idx, d):
    mesh = plsc.VectorSubcoreMesh(core_axis_name="c", subcore_axis_name="s", num_cores=NC, num_subcores=NS)
    idx_per_w = SEG_W * K
    nvec = d // 16

    @functools.partial(
        pl.kernel, mesh=mesh,
        compiler_params=pltpu.CompilerParams(use_tc_tiling_on_sc=False),
        out_type=jax.ShapeDtypeStruct((L_pad, d), jnp.float32),
        scratch_types=[
            pltpu.VMEM((idx_per_w,), jnp.int32),
            pltpu.VMEM((NBUF, SEG_CH * K, d), jnp.float32),
            pltpu.VMEM((SEG_W, d), jnp.float32),
            pltpu.SemaphoreType.DMA,
            pltpu.SemaphoreType.DMA,
            pltpu.SemaphoreType.DMA,
            pltpu.SemaphoreType.DMA,
        ],
    )
    def k(table_hbm, idx_hbm, out_hbm, idx_v, rows_v, acc_v, s0, s1, s2, s3):
        sems = (s0, s1, s2, s3)
        wid = _worker_id()
        pltpu.sync_copy(idx_hbm.at[pl.ds(wid * idx_per_w, idx_per_w)], idx_v)
        nch_s = SEG_W // SEG_CH

        def gather(c, buf):
            pltpu.async_copy(
                table_hbm.at[idx_v.at[pl.ds(c * SEG_CH * K, SEG_CH * K)]],
                rows_v.at[buf], sems[buf])

        for b in range(NBUF):
            gather(b, b)

        def step(g, carry):
            for b in range(NBUF):
                c = g * NBUF + b
                pltpu.make_async_copy(
                    table_hbm.at[idx_v.at[pl.ds(c * SEG_CH * K, SEG_CH * K)]],
                    rows_v.at[b], sems[b]).wait()
                for s in range(SEG_CH):
                    for j in range(nvec):
                        parts = [rows_v[b, s * K + p, pl.ds(j * 16, 16)]
                                 for p in range(4)]
                        for kk in range(4, K):
                            parts[kk % 4] = parts[kk % 4] + rows_v[
                                b, s * K + kk, pl.ds(j * 16, 16)]
                        acc_v[c * SEG_CH + s, pl.ds(j * 16, 16)] = (
                            (parts[0] + parts[1]) + (parts[2] + parts[3]))

                @pl.when(c + NBUF < nch_s)
                def _():
                    gather(c + NBUF, b)
            return carry

        lax.fori_loop(0, nch_s // NBUF, step, 0)
        pltpu.sync_copy(acc_v, out_hbm.at[pl.ds(wid * SEG_W, SEG_W)])

    return k(table, idx)


def _gelu(x):
    return 0.5 * x * (1.0 + lax.erf(x * 0.7071067811865476))


def tc_path_embed(pf_p, W1, b1, W2, b2):
    def body(pf_ref, w1_ref, b1_ref, w2_ref, b2_ref, out_ref):
        h = _gelu(jnp.dot(pf_ref[...], w1_ref[...],
                          preferred_element_type=jnp.float32) + b1_ref[...])
        out_ref[...] = _gelu(jnp.dot(h, w2_ref[...],
                                     preferred_element_type=jnp.float32) + b2_ref[...])

    bf = 3136
    grid = (F_pad // 2 // bf,)
    return pl.pallas_call(
        body,
        grid=grid,
        in_specs=[
            pl.BlockSpec((bf, 10), lambda i: (i, 0)),
            pl.BlockSpec((10, 2 * D), lambda i: (0, 0)),
            pl.BlockSpec((1, 2 * D), lambda i: (0, 0)),
            pl.BlockSpec((2 * D, 2 * D), lambda i: (0, 0)),
            pl.BlockSpec((1, 2 * D), lambda i: (0, 0)),
        ],
        out_specs=pl.BlockSpec((bf, 2 * D), lambda i: (i, 0)),
        out_shape=jax.ShapeDtypeStruct((F_pad // 2, 2 * D), jnp.float32),
    )(pf_p, W1, b1, W2, b2)


def tc_link_embed(cap, load_col, W1, b1, W2, b2):
    def body(cap_ref, load_ref, w1_ref, b1_ref, w2_ref, b2_ref, out_ref):
        c = cap_ref[...]
        f0 = c * 1e-05
        f1 = load_ref[...] / (c * 1e9)
        h = _gelu(f0 * w1_ref[0, :] + f1 * w1_ref[1, :] + b1_ref[...])
        out_ref[...] = _gelu(jnp.dot(h, w2_ref[...],
                                     preferred_element_type=jnp.float32) + b2_ref[...])

    return pl.pallas_call(
        body,
        grid=(1,),
        in_specs=[
            pl.BlockSpec((L_pad, 1), lambda i: (0, 0)),
            pl.BlockSpec((L_pad, 1), lambda i: (0, 0)),
            pl.BlockSpec((2, D), lambda i: (0, 0)),
            pl.BlockSpec((1, D), lambda i: (0, 0)),
            pl.BlockSpec((D, D), lambda i: (0, 0)),
            pl.BlockSpec((1, D), lambda i: (0, 0)),
        ],
        out_specs=pl.BlockSpec((L_pad, D), lambda i: (0, 0)),
        out_shape=jax.ShapeDtypeStruct((L_pad, D), jnp.float32),
    )(cap, load_col, W1, b1, W2, b2)


def _sigmoid(x):
    return 0.5 * (1.0 + jnp.tanh(0.5 * x))


def tc_gru_scan(xs_p, h0_p, Wzr, Whh, bs):
    bz, br_, bih, brh = bs
    P = 2 * D

    def body(xs_ref, h0_ref, wzr, whh, bz_r, br_r, bih_r, brh_r,
             out_ref, hl_ref):
        h = h0_ref[...]
        out_ref[0] = h
        for t in range(T):
            cat = jnp.concatenate([xs_ref[t], h], axis=1)
            zr = jnp.dot(cat, wzr[...], preferred_element_type=jnp.float32)
            xhh = jnp.dot(cat, whh[...], preferred_element_type=jnp.float32)
            z = _sigmoid(zr[:, :P] + bz_r[...])
            r = _sigmoid(zr[:, P:] + br_r[...])
            cand = jnp.tanh(xhh[:, :P] + bih_r[...] + r * (xhh[:, P:] + brh_r[...]))
            h = z * h + (1.0 - z) * cand
            out_ref[t + 1] = h
        hl_ref[...] = h

    bf = 784
    grid = (F_pad // 2 // bf,)
    wspec = pl.BlockSpec((2 * P, 2 * P), lambda i: (0, 0))
    bspec = pl.BlockSpec((1, P), lambda i: (0, 0))
    return pl.pallas_call(
        body,
        grid=grid,
        in_specs=[
            pl.BlockSpec((T, bf, P), lambda i: (0, i, 0)),
            pl.BlockSpec((bf, P), lambda i: (i, 0)),
            wspec, wspec,
            bspec, bspec, bspec, bspec,
        ],
        out_specs=[
            pl.BlockSpec((T + 1, bf, P), lambda i: (0, i, 0)),
            pl.BlockSpec((bf, P), lambda i: (i, 0)),
        ],
        out_shape=[
            jax.ShapeDtypeStruct((T + 1, F_pad // 2, P), jnp.float32),
            jax.ShapeDtypeStruct((F_pad // 2, P), jnp.float32),
        ],
    )(xs_p, h0_p, Wzr, Whh, *bs)


def tc_link_gru(ls, psum, Wzr, Whh, bs):
    def body(x_ref, h_ref, wzr, whh, bz_r, br_r, bih_r, brh_r, out_ref):
        h = h_ref[...]
        cat = jnp.concatenate([x_ref[...], h], axis=1)
        zr = jnp.dot(cat, wzr[...], preferred_element_type=jnp.float32)
        xhh = jnp.dot(cat, whh[...], preferred_element_type=jnp.float32)
        z = _sigmoid(zr[:, :D] + bz_r[...])
        r = _sigmoid(zr[:, D:] + br_r[...])
        cand = jnp.tanh(xhh[:, :D] + bih_r[...] + r * (xhh[:, D:] + brh_r[...]))
        out_ref[...] = z * h + (1.0 - z) * cand

    bl = L_pad // 4
    wspec = pl.BlockSpec((2 * D, 2 * D), lambda i: (0, 0))
    bspec = pl.BlockSpec((1, D), lambda i: (0, 0))
    return pl.pallas_call(
        body,
        grid=(L_pad // bl,),
        in_specs=[
            pl.BlockSpec((bl, D), lambda i: (i, 0)),
            pl.BlockSpec((bl, D), lambda i: (i, 0)),
            wspec, wspec,
            bspec, bspec, bspec, bspec,
        ],
        out_specs=pl.BlockSpec((bl, D), lambda i: (i, 0)),
        out_shape=jax.ShapeDtypeStruct((L_pad, D), jnp.float32),
    )(psum, ls, Wzr, Whh, *bs)


def _softplus(x):
    return jnp.maximum(x, 0.0) + jnp.log(1.0 + jnp.exp(-jnp.abs(x)))


def tc_readout(pss_p, cap16, W1, b1, W2, b2, W3, b3):
    def body(pss_ref, cap_ref, w1, b1r, w2, b2r, w3, b3r,
             outE_ref, outO_ref):
        cap = cap_ref[...]
        acc = None
        for t in range(T):
            x = pss_ref[t + 1]
            h1 = _gelu(jnp.dot(x, w1[...], preferred_element_type=jnp.float32) + b1r[...])
            h2 = _gelu(jnp.dot(h1, w2[...], preferred_element_type=jnp.float32) + b2r[...])
            occ = _softplus(jnp.dot(h2, w3[...], preferred_element_type=jnp.float32) + b3r[...])
            c = occ / cap[:, 2 * t:2 * t + 2]
            acc = c if acc is None else acc + c
        outE_ref[...] = acc[:, 0:1]
        outO_ref[...] = acc[:, 1:2]

    bf = 1568
    return pl.pallas_call(
        body,
        grid=(F_pad // 2 // bf,),
        in_specs=[
            pl.BlockSpec((T + 1, bf, 2 * D), lambda i: (0, i, 0)),
            pl.BlockSpec((bf, 2 * T), lambda i: (i, 0)),
            pl.BlockSpec((2 * D, D), lambda i: (0, 0)),
            pl.BlockSpec((1, D), lambda i: (0, 0)),
            pl.BlockSpec((D, D // 2), lambda i: (0, 0)),
            pl.BlockSpec((1, D // 2), lambda i: (0, 0)),
            pl.BlockSpec((D // 2, 2), lambda i: (0, 0)),
            pl.BlockSpec((1, 2), lambda i: (0, 0)),
        ],
        out_specs=[
            pl.BlockSpec((bf, 1), lambda i: (i, 0)),
            pl.BlockSpec((bf, 1), lambda i: (i, 0)),
        ],
        out_shape=[
            jax.ShapeDtypeStruct((F_pad // 2, 1), jnp.float32),
            jax.ShapeDtypeStruct((F_pad // 2, 1), jnp.float32),
        ],
    )(pss_p, cap16, W1, b1, W2, b2, W3, b3)


def _bd(A, B=None):
    if B is None:
        B = A
    n, m = A.shape
    p, q = B.shape
    return jnp.block([[A, jnp.zeros((n, q), jnp.float32)],
                      [jnp.zeros((p, m), jnp.float32), B]])


def _tile2(b):
    return jnp.concatenate([b, b]).reshape(1, -1)


def _split_gru(Kw, RK, bi, br):
    Kz, Kr, Kh = Kw[:, 0:D], Kw[:, D:2 * D], Kw[:, 2 * D:]
    Rz, Rr, Rh = RK[:, 0:D], RK[:, D:2 * D], RK[:, 2 * D:]
    Wzr_p = jnp.concatenate([
        jnp.concatenate([_bd(Kz), _bd(Kr)], axis=1),
        jnp.concatenate([_bd(Rz), _bd(Rr)], axis=1)], axis=0)
    zero = jnp.zeros((2 * D, 2 * D), jnp.float32)
    Whh_p = jnp.concatenate([
        jnp.concatenate([_bd(Kh), zero], axis=1),
        jnp.concatenate([zero, _bd(Rh)], axis=1)], axis=0)
    bs = (_tile2(bi[0:D] + br[0:D]),
          _tile2(bi[D:2 * D] + br[D:2 * D]),
          _tile2(bi[2 * D:]),
          _tile2(br[2 * D:]))
    return Wzr_p, Whh_p, bs


def _split_gru_unpaired(Kw, RK, bi, br):
    z = jnp.zeros((D, D), jnp.float32)
    Wzr = jnp.concatenate([Kw[:, :2 * D], RK[:, :2 * D]], axis=0)
    Whh = jnp.concatenate([
        jnp.concatenate([Kw[:, 2 * D:], z], axis=1),
        jnp.concatenate([z, RK[:, 2 * D:]], axis=1)], axis=0)
    bs = ((bi[0:D] + br[0:D]).reshape(1, D),
          (bi[D:2 * D] + br[D:2 * D]).reshape(1, D),
          bi[2 * D:].reshape(1, D),
          br[2 * D:].reshape(1, D))
    return Wzr, Whh, bs


def kernel(flow_traffic, flow_packets, flow_packet_size, link_capacity,
           ipg_mean, ipg_var, link_to_path, path_to_link,
           pe_W1, pe_b1, pe_W2, pe_b2, le_W1, le_b1, le_W2, le_b2,
           pg_K, pg_RK, pg_bi, pg_br, lg_K, lg_RK, lg_bi, lg_br,
           ro_W1, ro_b1, ro_W2, ro_b2, ro_W3, ro_b3):
    lt = link_to_path.astype(jnp.int32)
    idx_link = jnp.pad(lt.T, ((0, 0), (0, F_pad - F))).reshape(-1)

    p2l = path_to_link.astype(jnp.int32)
    flow = jnp.pad(p2l[:, :, 0], ((0, L_pad - L), (0, 0)))
    pos = jnp.pad(p2l[:, :, 1], ((0, L_pad - L), (0, 0)))
    idx_ps = (pos * F_pad + flow).reshape(-1)
    idx_traffic = flow.reshape(-1)

    traffic_tbl = jnp.broadcast_to(flow_traffic, (F, 16))
    cap_tbl = jnp.broadcast_to(link_capacity, (L, 16))

    load_sum = sc_gather_sum(traffic_tbl, idx_traffic, 16)

    scale = jnp.array(MM_SCALE, jnp.float32)[:, None]
    pf = jnp.pad(jnp.concatenate([flow_traffic, flow_packets, flow_packet_size,
                                  ipg_mean, ipg_var], axis=1),
                 ((0, F_pad - F), (0, 0)))
    pf_p = pf.reshape(F_pad // 2, 10)
    path_state = tc_path_embed(pf_p, _bd(pe_W1 * scale), _tile2(pe_b1),
                               _bd(pe_W2), _tile2(pe_b2))

    cap_pad = jnp.pad(link_capacity, ((0, L_pad - L), (0, 0)),
                      constant_values=1.0)
    link_state = tc_link_embed(cap_pad, load_sum[:, 0:1],
                               le_W1, le_b1.reshape(1, D),
                               le_W2, le_b2.reshape(1, D))

    capg = sc_gather(cap_tbl, idx_link, 16)
    cap3 = capg.reshape(T, F_pad, 16)[:, :, 0]
    cap16 = cap3.T.reshape(F_pad // 2, 2, T).transpose(0, 2, 1).reshape(
        F_pad // 2, 2 * T)

    pg_Wzr, pg_Whh, pg_bs = _split_gru(pg_K, pg_RK, pg_bi, pg_br)
    lg_Wzr, lg_Whh, lg_bs = _split_gru_unpaired(lg_K, lg_RK, lg_bi, lg_br)

    pss_p = None
    for r in range(4):
        xs = sc_gather(link_state, idx_link, D)
        xs_p = xs.reshape(T, F_pad // 2, 2 * D)
        pss_p, path_state = tc_gru_scan(xs_p, path_state,
                                        pg_Wzr, pg_Whh, pg_bs)
        if r < 3:
            psum = sc_gather_sum(pss_p.reshape((T + 1) * F_pad, D), idx_ps, D)
            link_state = tc_link_gru(link_state, psum, lg_Wzr, lg_Whh, lg_bs)

    dE, dO = tc_readout(pss_p, cap16,
                        _bd(ro_W1), _tile2(ro_b1),
                        _bd(ro_W2), _tile2(ro_b2),
                        _bd(ro_W3), _tile2(ro_b3))
    delay = jnp.stack([dE[:, 0], dO[:, 0]], axis=1).reshape(F_pad, 1)
    return delay[:F]
